# baseline (speedup 1.0000x reference)
import functools

import jax
import jax.numpy as jnp
from jax import lax
from jax.experimental import pallas as pl
from jax.experimental.pallas import tpu as pltpu
from jax.experimental.pallas import tpu_sc as plsc

HIDDEN = 32
EPS = 1e-5

NC = 2
NS = 16
NW = NC * NS
L = 16

CHUNK = 1024
SUB = 128
NSUB = CHUNK // SUB
GROUPS = CHUNK // L


def _rsqrt(x):
    i = plsc.bitcast(x, jnp.int32)
    i = jnp.int32(0x5F3759DF) - (i >> 1)
    y = plsc.bitcast(i, jnp.float32)
    for _ in range(3):
        y = y * (1.5 - 0.5 * x * y * y)
    return y


def _body(ids_hbm, table_hbm, gamma_hbm, beta_hbm, out_hbm,
          idx_v, rows_v, gamma_v, beta_v, gsem):
    wid = lax.axis_index("s") * NC + lax.axis_index("c")

    pltpu.sync_copy(gamma_hbm, gamma_v)
    pltpu.sync_copy(beta_hbm, beta_v)

    iota = lax.iota(jnp.int32, L)
    rows_2d = rows_v

    def chunk_body(g, carry):
        pltpu.sync_copy(ids_hbm.at[wid, g], idx_v)
        descs = [
            pltpu.async_copy(
                table_hbm.at[idx_v.at[j]],
                rows_2d.at[pl.ds(j * SUB, SUB)],
                gsem,
            )
            for j in range(NSUB)
        ]
        for d in descs:
            d.wait()

        def group_body(t, c2):
            row_ids = t * L + iota
            xs = []
            ssum = jnp.zeros((L,), jnp.float32)
            ssq = jnp.zeros((L,), jnp.float32)
            for h in range(HIDDEN):
                x = plsc.load_gather(rows_2d, [row_ids, jnp.full((L,), h, jnp.int32)])
                xs.append(x)
                ssum = ssum + x
                ssq = ssq + x * x
            mean = ssum * (1.0 / HIDDEN)
            var = ssq * (1.0 / HIDDEN) - mean * mean
            r = _rsqrt(var + EPS)
            for h in range(HIDDEN):
                gh = gamma_v[pl.ds(h * L, L)]
                bh = beta_v[pl.ds(h * L, L)]
                y = (xs[h] - mean) * r * gh + bh
                plsc.store_scatter(rows_2d, [row_ids, jnp.full((L,), h, jnp.int32)], y)
            return c2

        lax.fori_loop(0, GROUPS, group_body, 0)
        pltpu.sync_copy(rows_2d, out_hbm.at[wid, g])
        return carry

    lax.fori_loop(0, ids_hbm.shape[1], chunk_body, 0)


def kernel(input_ids, table, gamma, beta):
    B, S = input_ids.shape
    n = B * S
    assert n % (NW * CHUNK) == 0
    nchunk = n // (NW * CHUNK)

    ids = input_ids.astype(jnp.int32).reshape(NW, nchunk, NSUB, SUB)
    gamma_rep = jnp.broadcast_to(gamma[:, None], (HIDDEN, L)).reshape(-1)
    beta_rep = jnp.broadcast_to(beta[:, None], (HIDDEN, L)).reshape(-1)

    mesh = plsc.VectorSubcoreMesh(core_axis_name="c", subcore_axis_name="s")
    run = functools.partial(
        pl.kernel,
        out_type=jax.ShapeDtypeStruct((NW, nchunk, CHUNK, HIDDEN), jnp.float32),
        mesh=mesh,
        compiler_params=pltpu.CompilerParams(needs_layout_passes=False, use_tc_tiling_on_sc=False),
        scratch_types=[
            pltpu.VMEM((NSUB, SUB), jnp.int32),
            pltpu.VMEM((CHUNK, HIDDEN), jnp.float32),
            pltpu.VMEM((HIDDEN * L,), jnp.float32),
            pltpu.VMEM((HIDDEN * L,), jnp.float32),
            pltpu.SemaphoreType.DMA,
        ],
    )(_body)
    out = run(ids, table, gamma_rep, beta_rep)
    return out.reshape(B, S, HIDDEN)


# speedup vs baseline: 1.9769x; 1.9769x over previous
import functools

import jax
import jax.numpy as jnp
from jax import lax
from jax.experimental import pallas as pl
from jax.experimental.pallas import tpu as pltpu
from jax.experimental.pallas import tpu_sc as plsc

HIDDEN = 32
EPS = 1e-5

NC = 2
NS = 16
NW = NC * NS
L = 16

CHUNK = 1024
SUB = 128
NSUB = CHUNK // SUB
GROUPS = CHUNK // L
BANDS = HIDDEN // 8


def _rsqrt(x):
    i = plsc.bitcast(x, jnp.int32)
    i = jnp.int32(0x5F3759DF) - (i >> 1)
    y = plsc.bitcast(i, jnp.float32)
    for _ in range(3):
        y = y * (1.5 - 0.5 * x * y * y)
    return y


def _body(ids_hbm, table_hbm, gamma_hbm, beta_hbm, out_hbm,
          idx_v, rows_v, stage_v, gamma_v, beta_v, gsem):
    wid = lax.axis_index("s") * NC + lax.axis_index("c")

    pltpu.sync_copy(gamma_hbm, gamma_v)
    pltpu.sync_copy(beta_hbm, beta_v)

    iota = lax.iota(jnp.int32, L)
    nbq = ids_hbm.shape[1] // NSUB
    ntask = ids_hbm.shape[0] * nbq // NW

    def task_body(i, carry):
        t_id = wid * ntask + i
        s = t_id // nbq
        bq = t_id % nbq

        pltpu.sync_copy(ids_hbm.at[s, pl.ds(bq * NSUB, NSUB)], idx_v)
        descs = [
            pltpu.async_copy(
                table_hbm.at[idx_v.at[j]],
                rows_v.at[pl.ds(j * SUB, SUB)],
                gsem,
            )
            for j in range(NSUB)
        ]
        for d in descs:
            d.wait()

        def group_body(t, c2):
            row_ids = t * L + iota
            xs = []
            acc = [jnp.zeros((L,), jnp.float32) for _ in range(4)]
            acc2 = [jnp.zeros((L,), jnp.float32) for _ in range(4)]
            for h in range(HIDDEN):
                x = plsc.load_gather(rows_v, [row_ids, jnp.full((L,), h, jnp.int32)])
                xs.append(x)
                acc[h % 4] = acc[h % 4] + x
                acc2[h % 4] = acc2[h % 4] + x * x
            ssum = (acc[0] + acc[1]) + (acc[2] + acc[3])
            ssq = (acc2[0] + acc2[1]) + (acc2[2] + acc2[3])
            mean = ssum * (1.0 / HIDDEN)
            var = ssq * (1.0 / HIDDEN) - mean * mean
            r = _rsqrt(var + EPS)
            bt = t // 8
            bo = (t % 8) * L
            for h in range(HIDDEN):
                gh = gamma_v[pl.ds(h * L, L)]
                bh = beta_v[pl.ds(h * L, L)]
                y = (xs[h] - mean) * r * gh + bh
                hb, hh = h // 8, h % 8
                off = bt * (8 * SUB) + hh * SUB + bo
                stage_v[hb, pl.ds(off, L)] = y
            return c2

        lax.fori_loop(0, GROUPS, group_body, 0)
        for hb in range(BANDS):
            pltpu.sync_copy(
                stage_v.at[hb],
                out_hbm.at[s, hb, pl.ds(bq * NSUB * 8 * SUB, NSUB * 8 * SUB)],
            )
        return carry

    lax.fori_loop(0, ntask, task_body, 0)


def kernel(input_ids, table, gamma, beta):
    B, S = input_ids.shape
    n = B * S
    assert n % (NW * CHUNK) == 0 and B % SUB == 0

    ids = input_ids.astype(jnp.int32).T.reshape(S, B // SUB, SUB)
    gamma_rep = jnp.broadcast_to(gamma[:, None], (HIDDEN, L)).reshape(-1)
    beta_rep = jnp.broadcast_to(beta[:, None], (HIDDEN, L)).reshape(-1)

    mesh = plsc.VectorSubcoreMesh(core_axis_name="c", subcore_axis_name="s")
    run = functools.partial(
        pl.kernel,
        out_type=jax.ShapeDtypeStruct((S, BANDS, (B // SUB) * 8 * SUB), jnp.float32),
        mesh=mesh,
        compiler_params=pltpu.CompilerParams(
            needs_layout_passes=False, use_tc_tiling_on_sc=False),
        scratch_types=[
            pltpu.VMEM((NSUB, SUB), jnp.int32),
            pltpu.VMEM((CHUNK, HIDDEN), jnp.float32),
            pltpu.VMEM((BANDS, NSUB * 8 * SUB), jnp.float32),
            pltpu.VMEM((HIDDEN * L,), jnp.float32),
            pltpu.VMEM((HIDDEN * L,), jnp.float32),
            pltpu.SemaphoreType.DMA,
        ],
    )(_body)
    out = run(ids, table, gamma_rep, beta_rep)
    out = out.reshape(S, BANDS, B // SUB, 8, SUB)
    return out.transpose(2, 4, 0, 1, 3).reshape(B, S, HIDDEN)


# speedup vs baseline: 2.0228x; 1.0232x over previous
import functools

import jax
import jax.numpy as jnp
from jax import lax
from jax.experimental import pallas as pl
from jax.experimental.pallas import tpu as pltpu
from jax.experimental.pallas import tpu_sc as plsc

HIDDEN = 32
EPS = 1e-5

NC = 2
NS = 16
NW = NC * NS
L = 16

CHUNK = 512
SUB = 128
NSUB = CHUNK // SUB
GROUPS = CHUNK // L
BANDS = HIDDEN // 8
BSEG = NSUB * 8 * SUB


def _rsqrt(x):
    i = plsc.bitcast(x, jnp.int32)
    i = jnp.int32(0x5F3759DF) - (i >> 1)
    y = plsc.bitcast(i, jnp.float32)
    for _ in range(3):
        y = y * (1.5 - 0.5 * x * y * y)
    return y


def _body(ids_hbm, table_hbm, gamma_hbm, beta_hbm, out_hbm,
          idx_v, rows_v, stage_v, gamma_v, beta_v, sem0, sem1):
    wid = lax.axis_index("s") * NC + lax.axis_index("c")

    pltpu.sync_copy(gamma_hbm, gamma_v)
    pltpu.sync_copy(beta_hbm, beta_v)

    iota = lax.iota(jnp.int32, L)
    nbq = ids_hbm.shape[1] // NSUB
    ntask = ids_hbm.shape[0] * nbq // NW
    base = wid * ntask
    sems = (sem0, sem1)

    def start_gathers(i, buf):
        t_id = base + i
        s = t_id // nbq
        bq = t_id % nbq
        pltpu.sync_copy(ids_hbm.at[s, pl.ds(bq * NSUB, NSUB)], idx_v.at[buf])
        for j in range(NSUB):
            pltpu.async_copy(
                table_hbm.at[idx_v.at[buf, j]],
                rows_v.at[buf, pl.ds(j * SUB, SUB)],
                sems[buf],
            )

    def drain_gathers(buf):
        for j in range(NSUB):
            pltpu.make_async_copy(
                table_hbm.at[idx_v.at[buf, j]],
                rows_v.at[buf, pl.ds(j * SUB, SUB)],
                sems[buf],
            ).wait()

    def compute_and_flush(i, buf):
        rows = rows_v.at[buf]

        def group_body(t, c2):
            row_ids = t * L + iota
            xs = []
            acc = [jnp.zeros((L,), jnp.float32) for _ in range(4)]
            acc2 = [jnp.zeros((L,), jnp.float32) for _ in range(4)]
            for h in range(HIDDEN):
                x = plsc.load_gather(rows, [row_ids, jnp.full((L,), h, jnp.int32)])
                xs.append(x)
                acc[h % 4] = acc[h % 4] + x
                acc2[h % 4] = acc2[h % 4] + x * x
            ssum = (acc[0] + acc[1]) + (acc[2] + acc[3])
            ssq = (acc2[0] + acc2[1]) + (acc2[2] + acc2[3])
            mean = ssum * (1.0 / HIDDEN)
            var = ssq * (1.0 / HIDDEN) - mean * mean
            r = _rsqrt(var + EPS)
            bt = t // 8
            bo = (t % 8) * L
            for h in range(HIDDEN):
                gh = gamma_v[pl.ds(h * L, L)]
                bh = beta_v[pl.ds(h * L, L)]
                y = (xs[h] - mean) * r * gh + bh
                hb, hh = h // 8, h % 8
                off = bt * (8 * SUB) + hh * SUB + bo
                stage_v[hb, pl.ds(off, L)] = y
            return c2

        lax.fori_loop(0, GROUPS, group_body, 0)
        t_id = base + i
        s = t_id // nbq
        bq = t_id % nbq
        for hb in range(BANDS):
            pltpu.sync_copy(
                stage_v.at[hb],
                out_hbm.at[s, hb, pl.ds(bq * BSEG, BSEG)],
            )

    start_gathers(jnp.int32(0), 0)

    def pair_body(g, carry):
        for b in range(2):
            i = g * 2 + b

            @pl.when(i + 1 < ntask)
            def _():
                start_gathers(i + 1, (b + 1) % 2)

            drain_gathers(b)
            compute_and_flush(i, b)
        return carry

    lax.fori_loop(0, ntask // 2, pair_body, 0)


def kernel(input_ids, table, gamma, beta):
    B, S = input_ids.shape
    n = B * S
    assert n % (NW * CHUNK) == 0 and B % SUB == 0

    ids = input_ids.astype(jnp.int32).T.reshape(S, B // SUB, SUB)
    gamma_rep = jnp.broadcast_to(gamma[:, None], (HIDDEN, L)).reshape(-1)
    beta_rep = jnp.broadcast_to(beta[:, None], (HIDDEN, L)).reshape(-1)

    mesh = plsc.VectorSubcoreMesh(core_axis_name="c", subcore_axis_name="s")
    run = functools.partial(
        pl.kernel,
        out_type=jax.ShapeDtypeStruct((S, BANDS, (B // SUB) * 8 * SUB), jnp.float32),
        mesh=mesh,
        compiler_params=pltpu.CompilerParams(
            needs_layout_passes=False, use_tc_tiling_on_sc=False),
        scratch_types=[
            pltpu.VMEM((2, NSUB, SUB), jnp.int32),
            pltpu.VMEM((2, CHUNK, HIDDEN), jnp.float32),
            pltpu.VMEM((BANDS, BSEG), jnp.float32),
            pltpu.VMEM((HIDDEN * L,), jnp.float32),
            pltpu.VMEM((HIDDEN * L,), jnp.float32),
            pltpu.SemaphoreType.DMA,
            pltpu.SemaphoreType.DMA,
        ],
    )(_body)
    out = run(ids, table, gamma_rep, beta_rep)
    out = out.reshape(S, BANDS, B // SUB, 8, SUB)
    return out.transpose(2, 4, 0, 1, 3).reshape(B, S, HIDDEN)
